# baseline (speedup 1.0000x reference)
import functools

import jax
import jax.numpy as jnp
from jax import lax
from jax.experimental import pallas as pl
from jax.experimental.pallas import tpu as pltpu
from jax.experimental.pallas import tpu_sc as plsc

VOCAB = 1000000
DIM = 64
B = 4096
L = 200
EPS = 1e-12

NC = 2
NS = 16
LANES = 16
NW = NC * NS
BTILE = B // NW
PADW = 128
KV = DIM // LANES
CT = DIM // 8
UNROLL = 4
LPU = 2
OSTR = BTILE + 1


def _i16(v):
    return jnp.full((LANES,), v, jnp.int32)


def _rsqrt(x):
    i = lax.bitcast_convert_type(x, jnp.int32)
    y = lax.bitcast_convert_type(
        jnp.int32(0x5F3759DF) - lax.shift_right_logical(i, 1), jnp.float32)
    for _ in range(3):
        y = y * (1.5 - 0.5 * x * y * y)
    return y


def _sc_body(ids_hbm, table_hbm, w_hbm, b_hbm, out_hbm,
             ids_v, rows0, rows1, outt0, outt1, icol0, icol1,
             w_v, b_v, sem_g0, sem_g1, sem_o0, sem_o1):
    wkr = lax.axis_index("s") * NC + lax.axis_index("c")

    pltpu.sync_copy(ids_hbm.at[pl.ds(wkr * BTILE, BTILE)], ids_v)
    pltpu.sync_copy(w_hbm, w_v)
    pltpu.sync_copy(b_hbm, b_v)

    iota = lax.iota(jnp.int32, LANES)
    inv_dim = jnp.float32(1.0 / DIM)
    ct_half = lax.shift_right_logical(iota, 3)
    cc_lane = lax.bitwise_and(iota, _i16(7))

    def extract_idx(u, icol):
        for half in range(LPU):
            lv = jnp.zeros((LANES,), jnp.int32) + (LPU * u + half)
            for g in range(BTILE // LANES):
                v = plsc.load_gather(ids_v, [g * LANES + iota, lv])
                icol[half, pl.ds(g * LANES, LANES)] = v + v

    def fire_gather(icol, rows, sem):
        for half in range(LPU):
            pltpu.async_copy(table_hbm.at[icol.at[half]],
                             rows.at[pl.ds(half * BTILE, BTILE)], sem)

    def wait_gather(icol, rows, sem):
        for half in range(LPU):
            pltpu.make_async_copy(table_hbm.at[icol.at[half]],
                                  rows.at[pl.ds(half * BTILE, BTILE)],
                                  sem).wait()

    def compute(rows, outt):
        wb = ([w_v[pl.ds(k * LANES, LANES)] for k in range(KV)]
              + [b_v[pl.ds(k * LANES, LANES)] for k in range(KV)])

        def norm_body(un, wb):
            for tt in range(UNROLL):
                t = un * UNROLL + tt
                li = lax.shift_right_logical(t, 7)
                tb = lax.bitwise_and(t, BTILE - 1)
                vs = [rows[t, pl.ds(k * LANES, LANES)] for k in range(KV)]
                s = (vs[0] + vs[1]) + (vs[2] + vs[3])
                sq = (vs[0] * vs[0] + vs[1] * vs[1]) + (vs[2] * vs[2]
                                                        + vs[3] * vs[3])
                mean = jnp.sum(s) * inv_dim
                msq = jnp.sum(sq) * inv_dim
                var = msq - mean * mean
                rstd = _rsqrt(jnp.maximum(var, 0.0) + jnp.float32(EPS))
                c = -(mean * rstd)
                lv = jnp.zeros((LANES,), jnp.int32) + li
                tv = jnp.zeros((LANES,), jnp.int32) + tb
                for k in range(KV):
                    o = (vs[k] * rstd + c) * wb[k] + wb[KV + k]
                    plsc.store_scatter(
                        outt, [lv, 2 * k + ct_half, cc_lane, tv], o)
            return wb

        lax.fori_loop(0, LPU * BTILE // UNROLL, norm_body, tuple(wb))

    def fire_out(u, outt, sem):
        pltpu.async_copy(outt.at[:, :, :, pl.ds(0, BTILE)],
                         out_hbm.at[pl.ds(LPU * u, LPU), :, wkr], sem)

    def wait_out(outt, sem):
        pltpu.make_async_copy(outt.at[:, :, :, pl.ds(0, BTILE)],
                              out_hbm.at[pl.ds(0, LPU), :, wkr], sem).wait()

    extract_idx(0, icol0)
    fire_gather(icol0, rows0, sem_g0)
    extract_idx(1, icol1)
    fire_gather(icol1, rows1, sem_g1)

    NU = L // LPU

    def body(h, _):
        u0 = 2 * h
        u1 = 2 * h + 1

        @pl.when(h > 0)
        def _():
            wait_out(outt0, sem_o0)
        wait_gather(icol0, rows0, sem_g0)
        compute(rows0, outt0)
        fire_out(u0, outt0, sem_o0)

        @pl.when(h < NU // 2 - 1)
        def _():
            extract_idx(u0 + 2, icol0)
            fire_gather(icol0, rows0, sem_g0)

        @pl.when(h > 0)
        def _():
            wait_out(outt1, sem_o1)
        wait_gather(icol1, rows1, sem_g1)
        compute(rows1, outt1)
        fire_out(u1, outt1, sem_o1)

        @pl.when(h < NU // 2 - 1)
        def _():
            extract_idx(u1 + 2, icol1)
            fire_gather(icol1, rows1, sem_g1)
        return 0

    lax.fori_loop(0, NU // 2, body, 0)
    wait_out(outt0, sem_o0)
    wait_out(outt1, sem_o1)


@jax.jit
def _sc_embed_ln(ids, table_pad, ln_weight, ln_bias):
    mesh = plsc.VectorSubcoreMesh(
        core_axis_name="c", subcore_axis_name="s",
        num_cores=NC, num_subcores=NS)
    return pl.kernel(
        _sc_body,
        out_type=jax.ShapeDtypeStruct((L, CT, NW, 8, 128), jnp.float32),
        mesh=mesh,
        compiler_params=pltpu.CompilerParams(
            needs_layout_passes=False, use_tc_tiling_on_sc=False),
        scratch_types=[
            pltpu.VMEM((BTILE, L), jnp.int32),
            pltpu.VMEM((LPU * BTILE, DIM), jnp.float32),
            pltpu.VMEM((LPU * BTILE, DIM), jnp.float32),
            pltpu.VMEM((LPU, CT, 8, OSTR), jnp.float32),
            pltpu.VMEM((LPU, CT, 8, OSTR), jnp.float32),
            pltpu.VMEM((LPU, BTILE), jnp.int32),
            pltpu.VMEM((LPU, BTILE), jnp.int32),
            pltpu.VMEM((DIM,), jnp.float32),
            pltpu.VMEM((DIM,), jnp.float32),
            pltpu.SemaphoreType.DMA,
            pltpu.SemaphoreType.DMA,
            pltpu.SemaphoreType.DMA,
            pltpu.SemaphoreType.DMA,
        ],
    )(ids, table_pad, ln_weight, ln_bias)


def kernel(input_ids, table, ln_weight, ln_bias):
    table_pad = jnp.pad(table, ((0, 0), (0, PADW - DIM)))
    table_view = table_pad.reshape(2 * VOCAB, DIM)
    out5 = _sc_embed_ln(input_ids.astype(jnp.int32), table_view,
                        ln_weight, ln_bias)
    return out5.transpose(2, 4, 0, 1, 3).reshape(B, L, DIM)
